# Initial kernel scaffold; baseline (speedup 1.0000x reference)
import jax
import jax.numpy as jnp
from jax.experimental import pallas as pl


def kernel(tokens, cat_W, num_W, token_W):
    raise NotImplementedError("write your pallas kernel here")


# speedup vs baseline: 1.3056x; 1.3056x over previous
import functools

import jax
import jax.numpy as jnp
from jax import lax
from jax.experimental import pallas as pl
from jax.experimental.pallas import tpu as pltpu
from jax.experimental.pallas import tpu_sc as plsc

BLOCK = 16
VOCAB = 1024
D = 768

_info = plsc.get_sparse_core_info()
NC, NS, L = _info.num_cores, _info.num_subcores, _info.num_lanes
NW = NC * NS


def _sc_body(tokens_hbm, cat_w_hbm, num_w_hbm, token_w_hbm,
             seq_hbm, cat_ids_hbm, hist_hbm,
             tok_v, rows_t, rows_n, rows_c, acc_v, hist_v, cat_v,
             sem_t, sem_n, sem_c):
    n_tok = tokens_hbm.shape[0]
    blocks_total = n_tok // BLOCK
    blk_per_w = blocks_total // NW
    n_blocks_per_batch = 256
    seq_rows_per_batch = 2 * n_blocks_per_batch + 4096

    wid = lax.axis_index("s") * NC + lax.axis_index("c")
    blk0 = wid * blk_per_w
    b = blk0 // n_blocks_per_batch
    n0 = blk0 - b * n_blocks_per_batch

    pltpu.sync_copy(tokens_hbm.at[pl.ds(blk0 * BLOCK, blk_per_w * BLOCK)],
                    tok_v)

    zeros16 = jnp.zeros((L,), jnp.float32)
    ones16 = jnp.ones((L,), jnp.float32)
    for i in range(VOCAB // L):
        hist_v[pl.ds(i * L, L)] = zeros16

    lane = lax.iota(jnp.int32, L)
    for h in range(blk_per_w // L):
        cat_idx = jnp.zeros((L,), jnp.int32)
        for k in range(L):
            t0 = tok_v[pl.ds((h * L + k) * BLOCK, L)][0]
            cat_idx = jnp.where(lane == k, t0, cat_idx)
        cat_v[pl.ds(h * L, L)] = cat_idx
        pltpu.async_copy(cat_w_hbm.at[cat_idx], rows_c, sem_c).wait()
        row0 = b * seq_rows_per_batch + n0 + h * L
        pltpu.sync_copy(rows_c, seq_hbm.at[pl.ds(row0, L)])
    pltpu.sync_copy(cat_v, cat_ids_hbm.at[pl.ds(wid * blk_per_w, blk_per_w)])

    def blk_body(j, _):
        tok_idx = tok_v[pl.ds(j * BLOCK, BLOCK)]
        cp_t = pltpu.async_copy(token_w_hbm.at[tok_idx], rows_t, sem_t)
        cp_n = pltpu.async_copy(num_w_hbm.at[tok_idx], rows_n, sem_n)

        plsc.addupdate_scatter(hist_v, [tok_idx], ones16)

        cp_t.wait()
        tok_row0 = (b * seq_rows_per_batch + 2 * n_blocks_per_batch
                    + (n0 + j) * BLOCK)
        pltpu.sync_copy(rows_t, seq_hbm.at[pl.ds(tok_row0, BLOCK)])

        cp_n.wait()

        def chunk_body(c, _):
            s = rows_n[0, pl.ds(c * L, L)]
            for r in range(1, BLOCK):
                s = s + rows_n[r, pl.ds(c * L, L)]
            acc_v[pl.ds(c * L, L)] = s
            return 0

        lax.fori_loop(0, D // L, chunk_body, 0)
        num_row = b * seq_rows_per_batch + n_blocks_per_batch + n0 + j
        pltpu.sync_copy(acc_v, seq_hbm.at[num_row])

        pltpu.sync_copy(hist_v, hist_hbm.at[blk0 + j])
        plsc.store_scatter(hist_v, [tok_idx], zeros16)
        return 0

    lax.fori_loop(0, blk_per_w, blk_body, 0)


def kernel(tokens, cat_W, num_W, token_W):
    B, Lseq = tokens.shape
    n_blocks = Lseq // BLOCK
    seq_rows = 2 * n_blocks + Lseq

    mesh = plsc.VectorSubcoreMesh(core_axis_name="c", subcore_axis_name="s")
    sc = pl.kernel(
        _sc_body,
        out_type=[
            jax.ShapeDtypeStruct((B * seq_rows, D), jnp.float32),
            jax.ShapeDtypeStruct((B * n_blocks,), jnp.int32),
            jax.ShapeDtypeStruct((B * n_blocks, VOCAB), jnp.float32),
        ],
        mesh=mesh,
        compiler_params=pltpu.CompilerParams(needs_layout_passes=False),
        scratch_types=[
            pltpu.VMEM((Lseq * B // NW,), jnp.int32),
            pltpu.VMEM((BLOCK, D), jnp.float32),
            pltpu.VMEM((BLOCK, D), jnp.float32),
            pltpu.VMEM((L, D), jnp.float32),
            pltpu.VMEM((D,), jnp.float32),
            pltpu.VMEM((VOCAB,), jnp.float32),
            pltpu.VMEM((B * n_blocks // NW,), jnp.int32),
            pltpu.SemaphoreType.DMA,
            pltpu.SemaphoreType.DMA,
            pltpu.SemaphoreType.DMA,
        ],
    )
    seq_flat, cat_ids_flat, hist_flat = sc(
        tokens.reshape(-1), cat_W, num_W, token_W)
    new_seq = seq_flat.reshape(B, seq_rows, D)
    cat_ids = cat_ids_flat.reshape(B, n_blocks)
    hist = hist_flat.reshape(B, n_blocks, VOCAB)
    return (new_seq, cat_ids, hist)


# speedup vs baseline: 1.9741x; 1.5120x over previous
import jax
import jax.numpy as jnp
from jax import lax
from jax.experimental import pallas as pl
from jax.experimental.pallas import tpu as pltpu
from jax.experimental.pallas import tpu_sc as plsc

BLOCK = 16
VOCAB = 1024
D = 768

_info = plsc.get_sparse_core_info()
NC, NS, L = _info.num_cores, _info.num_subcores, _info.num_lanes
NW = NC * NS

PAIR = 2 * BLOCK


def _sc_body(tokens_hbm, cat_w_hbm, num_w_hbm, token_w_hbm,
             seq_hbm, cat_ids_hbm, hist_hbm,
             tok_v, tokrows0, tokrows1, numrows0, numrows1,
             rows_c, acc0, acc1, hist0, hist1, cat_v,
             gt0, gt1, gn0, gn1, ot0, ot1, on0, on1, oh0, oh1, sem_c):
    tokrows = (tokrows0, tokrows1)
    numrows = (numrows0, numrows1)
    acc2 = (acc0, acc1)
    hist2 = (hist0, hist1)
    gt = (gt0, gt1)
    gn = (gn0, gn1)
    ot = (ot0, ot1)
    on = (on0, on1)
    oh = (oh0, oh1)

    n_tok = tokens_hbm.shape[0]
    blocks_total = n_tok // BLOCK
    blk_per_w = blocks_total // NW
    n_iters = blk_per_w // 2
    npb = 256
    spb = 2 * npb + 4096

    wid = lax.axis_index("s") * NC + lax.axis_index("c")
    blk0 = wid * blk_per_w
    b = blk0 // npb
    n0 = blk0 - b * npb

    zeros16 = jnp.zeros((L,), jnp.float32)
    ones16 = jnp.ones((L,), jnp.float32)
    lane = lax.iota(jnp.int32, L)

    pltpu.sync_copy(tokens_hbm.at[pl.ds(blk0 * BLOCK, blk_per_w * BLOCK)],
                    tok_v)

    for h in range(blk_per_w // L):
        cat_idx = jnp.zeros((L,), jnp.int32)
        for k in range(L):
            t0 = tok_v[pl.ds((h * L + k) * BLOCK, L)][0]
            cat_idx = jnp.where(lane == k, t0, cat_idx)
        cat_v[pl.ds(h * L, L)] = cat_idx
        pltpu.async_copy(cat_w_hbm.at[cat_idx], rows_c, sem_c).wait()
        row0 = b * spb + n0 + h * L
        pltpu.sync_copy(rows_c, seq_hbm.at[pl.ds(row0, L)])
    pltpu.sync_copy(cat_v, cat_ids_hbm.at[pl.ds(wid * blk_per_w, blk_per_w)])

    for p in (0, 1):
        for q in (0, 1):
            for i in range(VOCAB // L):
                hist2[p][q, pl.ds(i * L, L)] = zeros16

    def issue_gathers(k, p):
        idx = tok_v.at[pl.ds(k * PAIR, PAIR)]
        pltpu.async_copy(token_w_hbm.at[idx], tokrows[p], gt[p])
        pltpu.async_copy(num_w_hbm.at[idx], numrows[p], gn[p])

    def wait_gathers(p):
        pltpu.make_async_copy(token_w_hbm.at[pl.ds(0, PAIR)],
                              tokrows[p], gt[p]).wait()
        pltpu.make_async_copy(num_w_hbm.at[pl.ds(0, PAIR)],
                              numrows[p], gn[p]).wait()

    def drain_outs(p):
        pltpu.make_async_copy(tokrows[p], seq_hbm.at[pl.ds(0, PAIR)],
                              ot[p]).wait()
        pltpu.make_async_copy(acc2[p], seq_hbm.at[pl.ds(0, 2)], on[p]).wait()
        pltpu.make_async_copy(hist2[p], hist_hbm.at[pl.ds(0, 2)], oh[p]).wait()

    issue_gathers(0, 0)

    def outer(kk, _):
        for p in (0, 1):
            k = kk * 2 + p
            nk = k + 1
            @pl.when(jnp.logical_and(k >= 1, nk < n_iters))
            def _():
                drain_outs(1 - p)

            @pl.when(nk < n_iters)
            def _():
                issue_gathers(nk, 1 - p)

            wait_gathers(p)

            tok_row0 = b * spb + 2 * npb + (n0 + k * 2) * BLOCK
            pltpu.async_copy(tokrows[p], seq_hbm.at[pl.ds(tok_row0, PAIR)],
                             ot[p])

            @pl.when(k >= 2)
            def _():
                for q in (0, 1):
                    old_idx = tok_v[pl.ds(((k - 2) * 2 + q) * BLOCK, BLOCK)]
                    qv = jnp.full((L,), q, jnp.int32)
                    plsc.store_scatter(hist2[p], [qv, old_idx], zeros16)

            for q in (0, 1):
                tok_idx = tok_v[pl.ds((k * 2 + q) * BLOCK, BLOCK)]
                qv = jnp.full((L,), q, jnp.int32)
                plsc.addupdate_scatter(hist2[p], [qv, tok_idx], ones16)

                def chunk_body(c, _, q=q, p=p):
                    s = numrows[p][q * BLOCK, pl.ds(c * L, L)]
                    for r in range(1, BLOCK):
                        s = s + numrows[p][q * BLOCK + r, pl.ds(c * L, L)]
                    acc2[p][q, pl.ds(c * L, L)] = s
                    return 0

                lax.fori_loop(0, D // L, chunk_body, 0)

            num_row0 = b * spb + npb + n0 + k * 2
            pltpu.async_copy(acc2[p], seq_hbm.at[pl.ds(num_row0, 2)], on[p])
            pltpu.async_copy(hist2[p], hist_hbm.at[pl.ds(blk0 + k * 2, 2)],
                             oh[p])
        return 0

    lax.fori_loop(0, n_iters // 2, outer, 0)

    drain_outs(0)
    drain_outs(1)


def kernel(tokens, cat_W, num_W, token_W):
    B, Lseq = tokens.shape
    n_blocks = Lseq // BLOCK
    seq_rows = 2 * n_blocks + Lseq

    mesh = plsc.VectorSubcoreMesh(core_axis_name="c", subcore_axis_name="s")
    sc = pl.kernel(
        _sc_body,
        out_type=[
            jax.ShapeDtypeStruct((B * seq_rows, D), jnp.float32),
            jax.ShapeDtypeStruct((B * n_blocks,), jnp.int32),
            jax.ShapeDtypeStruct((B * n_blocks, VOCAB), jnp.float32),
        ],
        mesh=mesh,
        compiler_params=pltpu.CompilerParams(needs_layout_passes=False),
        scratch_types=[
            pltpu.VMEM((Lseq * B // NW,), jnp.int32),
            pltpu.VMEM((PAIR, D), jnp.float32),
            pltpu.VMEM((PAIR, D), jnp.float32),
            pltpu.VMEM((PAIR, D), jnp.float32),
            pltpu.VMEM((PAIR, D), jnp.float32),
            pltpu.VMEM((L, D), jnp.float32),
            pltpu.VMEM((2, D), jnp.float32),
            pltpu.VMEM((2, D), jnp.float32),
            pltpu.VMEM((2, VOCAB), jnp.float32),
            pltpu.VMEM((2, VOCAB), jnp.float32),
            pltpu.VMEM((B * n_blocks // NW,), jnp.int32),
        ] + [pltpu.SemaphoreType.DMA] * 11,
    )
    seq_flat, cat_ids_flat, hist_flat = sc(
        tokens.reshape(-1), cat_W, num_W, token_W)
    new_seq = seq_flat.reshape(B, seq_rows, D)
    cat_ids = cat_ids_flat.reshape(B, n_blocks)
    hist = hist_flat.reshape(B, n_blocks, VOCAB)
    return (new_seq, cat_ids, hist)
